# Initial kernel scaffold; baseline (speedup 1.0000x reference)
import functools

import jax
import jax.numpy as jnp
from jax import lax
from jax.experimental import pallas as pl
from jax.experimental.pallas import tpu as pltpu
from jax.experimental.pallas import tpu_sc as plsc

B = 1024
L = 200
D = 64
NC = 2
NS = 16
NW = NC * NS
BPW = B // NW
H = L // 2


def _sc_embed(dx_idx, proc_idx, dx_table, proc_table, pe_block):
    mesh = plsc.VectorSubcoreMesh(core_axis_name="c", subcore_axis_name="s")

    @functools.partial(
        pl.kernel,
        out_type=(
            jax.ShapeDtypeStruct((B * L, D), jnp.float32),
            jax.ShapeDtypeStruct((B * L, D), jnp.float32),
        ),
        mesh=mesh,
        scratch_types=[
            pltpu.VMEM((2, H), jnp.int32),
            pltpu.VMEM((2, H), jnp.int32),
            pltpu.VMEM((L, D), jnp.float32),
            pltpu.VMEM((L, D), jnp.float32),
            pltpu.VMEM((L, D), jnp.float32),
            pltpu.SemaphoreType.DMA,
            pltpu.SemaphoreType.DMA,
        ],
    )
    def k(dxi_hbm, pri_hbm, dxt_hbm, prt_hbm, pe_hbm,
          dx_out, pr_out, idx_dx, idx_pr, rows_dx, rows_pr, pe_v,
          sem_dx, sem_pr):
        wid = lax.axis_index("s") * NC + lax.axis_index("c")
        pltpu.sync_copy(pe_hbm, pe_v)

        def body(i, carry):
            b = wid * BPW + i
            pltpu.sync_copy(dxi_hbm.at[b], idx_dx)
            pltpu.sync_copy(pri_hbm.at[b], idx_pr)
            g0 = pltpu.async_copy(dxt_hbm.at[idx_dx.at[0]],
                                  rows_dx.at[pl.ds(0, H)], sem_dx)
            g1 = pltpu.async_copy(dxt_hbm.at[idx_dx.at[1]],
                                  rows_dx.at[pl.ds(H, H)], sem_dx)
            g2 = pltpu.async_copy(prt_hbm.at[idx_pr.at[0]],
                                  rows_pr.at[pl.ds(0, H)], sem_pr)
            g3 = pltpu.async_copy(prt_hbm.at[idx_pr.at[1]],
                                  rows_pr.at[pl.ds(H, H)], sem_pr)
            g0.wait()
            g1.wait()

            def add_dx(j, c):
                for q in range(D // 16):
                    sl = pl.ds(q * 16, 16)
                    plsc.addupdate(rows_dx.at[j, sl], pe_v[j, sl])
                return c

            lax.fori_loop(0, L, add_dx, 0)
            pltpu.sync_copy(rows_dx, dx_out.at[pl.ds(b * L, L)])
            g2.wait()
            g3.wait()

            def add_pr(j, c):
                for q in range(D // 16):
                    sl = pl.ds(q * 16, 16)
                    plsc.addupdate(rows_pr.at[j, sl], pe_v[j, sl])
                return c

            lax.fori_loop(0, L, add_pr, 0)
            pltpu.sync_copy(rows_pr, pr_out.at[pl.ds(b * L, L)])
            return carry

        lax.fori_loop(0, BPW, body, 0)

    return k(dx_idx, proc_idx, dx_table, proc_table, pe_block)


def kernel(dx_ints, proc_ints, dx_table, proc_table, visit_table, pe):
    dx_idx = dx_ints.astype(jnp.int32).reshape(B, 2, H)
    proc_idx = proc_ints.astype(jnp.int32).reshape(B, 2, H)
    pe_block = pe[:L, 0, :]
    dx_flat, pr_flat = _sc_embed(dx_idx, proc_idx, dx_table, proc_table,
                                 pe_block)
    dx_emb = dx_flat.reshape(B, L, D)
    proc_emb = pr_flat.reshape(B, L, D)
    visit = jnp.broadcast_to(visit_table[0][None, None, :], (B, 1, D))
    visit_mask = jnp.ones((B, 1), dtype=jnp.float32)
    return (dx_emb, proc_emb, visit, visit_mask)


# speedup vs baseline: 2.8684x; 2.8684x over previous
import functools

import jax
import jax.numpy as jnp
from jax import lax
from jax.experimental import pallas as pl
from jax.experimental.pallas import tpu as pltpu
from jax.experimental.pallas import tpu_sc as plsc

B = 1024
L = 200
D = 64
NC = 2
NS = 16
NW = NC * NS
BPW = B // NW
H = L // 2


def _sc_embed(dx_idx, proc_idx, dx_table, proc_table, pe_block):
    mesh = plsc.VectorSubcoreMesh(core_axis_name="c", subcore_axis_name="s")

    @functools.partial(
        pl.kernel,
        out_type=(
            jax.ShapeDtypeStruct((B * L, D), jnp.float32),
            jax.ShapeDtypeStruct((B * L, D), jnp.float32),
        ),
        mesh=mesh,
        compiler_params=pltpu.CompilerParams(use_tc_tiling_on_sc=False),
        scratch_types=[
            pltpu.VMEM((2, H), jnp.int32),
            pltpu.VMEM((2, H), jnp.int32),
            pltpu.VMEM((L, D), jnp.float32),
            pltpu.VMEM((L, D), jnp.float32),
            pltpu.VMEM((L, D), jnp.float32),
            pltpu.SemaphoreType.DMA,
            pltpu.SemaphoreType.DMA,
        ],
    )
    def k(dxi_hbm, pri_hbm, dxt_hbm, prt_hbm, pe_hbm,
          dx_out, pr_out, idx_dx, idx_pr, rows_dx, rows_pr, pe_v,
          sem_dx, sem_pr):
        wid = lax.axis_index("s") * NC + lax.axis_index("c")
        pltpu.sync_copy(pe_hbm, pe_v)

        def body(i, carry):
            b = wid * BPW + i
            pltpu.sync_copy(dxi_hbm.at[b], idx_dx)
            pltpu.sync_copy(pri_hbm.at[b], idx_pr)
            g0 = pltpu.async_copy(dxt_hbm.at[idx_dx.at[0]],
                                  rows_dx.at[pl.ds(0, H)], sem_dx)
            g1 = pltpu.async_copy(dxt_hbm.at[idx_dx.at[1]],
                                  rows_dx.at[pl.ds(H, H)], sem_dx)
            g2 = pltpu.async_copy(prt_hbm.at[idx_pr.at[0]],
                                  rows_pr.at[pl.ds(0, H)], sem_pr)
            g3 = pltpu.async_copy(prt_hbm.at[idx_pr.at[1]],
                                  rows_pr.at[pl.ds(H, H)], sem_pr)
            g0.wait()
            g1.wait()

            def add_dx(j, c):
                for q in range(D // 16):
                    sl = pl.ds(q * 16, 16)
                    plsc.addupdate(rows_dx.at[j, sl], pe_v[j, sl])
                return c

            lax.fori_loop(0, L, add_dx, 0)
            pltpu.sync_copy(rows_dx, dx_out.at[pl.ds(b * L, L)])
            g2.wait()
            g3.wait()

            def add_pr(j, c):
                for q in range(D // 16):
                    sl = pl.ds(q * 16, 16)
                    plsc.addupdate(rows_pr.at[j, sl], pe_v[j, sl])
                return c

            lax.fori_loop(0, L, add_pr, 0)
            pltpu.sync_copy(rows_pr, pr_out.at[pl.ds(b * L, L)])
            return carry

        lax.fori_loop(0, BPW, body, 0)

    return k(dx_idx, proc_idx, dx_table, proc_table, pe_block)


def kernel(dx_ints, proc_ints, dx_table, proc_table, visit_table, pe):
    dx_idx = dx_ints.astype(jnp.int32).reshape(B, 2, H)
    proc_idx = proc_ints.astype(jnp.int32).reshape(B, 2, H)
    pe_block = pe[:L, 0, :]
    dx_flat, pr_flat = _sc_embed(dx_idx, proc_idx, dx_table, proc_table,
                                 pe_block)
    dx_emb = dx_flat.reshape(B, L, D)
    proc_emb = pr_flat.reshape(B, L, D)
    visit = jnp.broadcast_to(visit_table[0][None, None, :], (B, 1, D))
    visit_mask = jnp.ones((B, 1), dtype=jnp.float32)
    return (dx_emb, proc_emb, visit, visit_mask)


# speedup vs baseline: 3.4979x; 1.2195x over previous
import functools

import jax
import jax.numpy as jnp
from jax import lax
from jax.experimental import pallas as pl
from jax.experimental.pallas import tpu as pltpu
from jax.experimental.pallas import tpu_sc as plsc

B = 1024
L = 200
D = 64
NC = 2
NS = 16
NW = NC * NS
BPW = B // NW
H = L // 2
NBUF = 3


def _sc_embed(dx_idx, proc_idx, dx_table, proc_table, pe_block):
    mesh = plsc.VectorSubcoreMesh(core_axis_name="c", subcore_axis_name="s")

    scratch = {
        "idx_dx": pltpu.VMEM((2 * BPW, H), jnp.int32),
        "idx_pr": pltpu.VMEM((2 * BPW, H), jnp.int32),
        "pe_v": pltpu.VMEM((L, D), jnp.float32),
        "rows": [pltpu.VMEM((L, D), jnp.float32) for _ in range(2 * NBUF)],
        "gsem": [pltpu.SemaphoreType.DMA for _ in range(NBUF)],
        "ssem": [pltpu.SemaphoreType.DMA for _ in range(NBUF)],
    }

    @functools.partial(
        pl.kernel,
        out_type=(
            jax.ShapeDtypeStruct((B * L, D), jnp.float32),
            jax.ShapeDtypeStruct((B * L, D), jnp.float32),
        ),
        mesh=mesh,
        compiler_params=pltpu.CompilerParams(use_tc_tiling_on_sc=False),
        scratch_types=scratch,
    )
    def k(dxi_hbm, pri_hbm, dxt_hbm, prt_hbm, pe_hbm, dx_out, pr_out,
          idx_dx, idx_pr, pe_v, rows, gsem, ssem):
        wid = lax.axis_index("s") * NC + lax.axis_index("c")
        pltpu.sync_copy(pe_hbm, pe_v)
        pltpu.sync_copy(dxi_hbm.at[wid], idx_dx)
        pltpu.sync_copy(pri_hbm.at[wid], idx_pr)

        def fire_gathers(i):
            s = i % NBUF
            return [
                pltpu.async_copy(dxt_hbm.at[idx_dx.at[2 * i]],
                                 rows[2 * s].at[pl.ds(0, H)], gsem[s]),
                pltpu.async_copy(dxt_hbm.at[idx_dx.at[2 * i + 1]],
                                 rows[2 * s].at[pl.ds(H, H)], gsem[s]),
                pltpu.async_copy(prt_hbm.at[idx_pr.at[2 * i]],
                                 rows[2 * s + 1].at[pl.ds(0, H)], gsem[s]),
                pltpu.async_copy(prt_hbm.at[idx_pr.at[2 * i + 1]],
                                 rows[2 * s + 1].at[pl.ds(H, H)], gsem[s]),
            ]

        def add_pe(buf):
            @plsc.parallel_loop(0, L, unroll=4)
            def _(j):
                for q in range(D // 16):
                    sl = pl.ds(q * 16, 16)
                    plsc.addupdate(buf.at[j, sl], pe_v[j, sl])

        gd = [None] * BPW
        sd = [None] * BPW
        gd[0] = fire_gathers(0)
        gd[1] = fire_gathers(1)
        for i in range(BPW):
            s = i % NBUF
            b = wid * BPW + i
            for g in gd[i]:
                g.wait()
            add_pe(rows[2 * s])
            st0 = pltpu.async_copy(rows[2 * s],
                                   dx_out.at[pl.ds(b * L, L)], ssem[s])
            add_pe(rows[2 * s + 1])
            st1 = pltpu.async_copy(rows[2 * s + 1],
                                   pr_out.at[pl.ds(b * L, L)], ssem[s])
            sd[i] = (st0, st1)
            if i + 2 < BPW:
                if i >= 1:
                    for st in sd[i - 1]:
                        st.wait()
                gd[i + 2] = fire_gathers(i + 2)
        for i in range(BPW - NBUF, BPW):
            for st in sd[i]:
                st.wait()

    return k(dx_idx, proc_idx, dx_table, proc_table, pe_block)


def kernel(dx_ints, proc_ints, dx_table, proc_table, visit_table, pe):
    dx_idx = dx_ints.astype(jnp.int32).reshape(NW, 2 * BPW, H)
    proc_idx = proc_ints.astype(jnp.int32).reshape(NW, 2 * BPW, H)
    pe_block = pe[:L, 0, :]
    dx_flat, pr_flat = _sc_embed(dx_idx, proc_idx, dx_table, proc_table,
                                 pe_block)
    dx_emb = dx_flat.reshape(B, L, D)
    proc_emb = pr_flat.reshape(B, L, D)
    visit = jnp.broadcast_to(visit_table[0][None, None, :], (B, 1, D))
    visit_mask = jnp.ones((B, 1), dtype=jnp.float32)
    return (dx_emb, proc_emb, visit, visit_mask)
